# speedup vs baseline: 5.3172x; 5.3172x over previous
import jax
import jax.numpy as jnp
from jax.experimental import pallas as pl
from jax.experimental.pallas import tpu as pltpu


def _router_body(x_ref, wr_ref, logits_ref, tkw_ref, tki_ref):
    x = x_ref[...]
    inv_l = 1.0 / x.shape[1]
    xm = jnp.sum(x, axis=1) * inv_l
    logits = jnp.dot(xm, wr_ref[...], preferred_element_type=jnp.float32)
    logits_ref[...] = logits
    p = jax.nn.softmax(logits, axis=-1)
    iota = jax.lax.broadcasted_iota(jnp.int32, p.shape, 1)
    i1 = jnp.argmax(p, axis=-1).astype(jnp.int32)
    m1 = jnp.max(p, axis=-1)
    p2 = jnp.where(iota == i1[:, None], -jnp.inf, p)
    i2 = jnp.argmax(p2, axis=-1).astype(jnp.int32)
    m2 = jnp.max(p2, axis=-1)
    tkw_ref[...] = jnp.concatenate([m1[:, None], m2[:, None]], axis=1)
    tki_ref[...] = jnp.concatenate([i1[:, None], i2[:, None]], axis=1)


def _moe_body(tki_ref, tkw_ref, x_ref, w1_ref, w3_ref, w2_ref, out_ref):
    b = pl.program_id(0)
    k = pl.program_id(1)
    f = pl.program_id(2)
    x = x_ref[0]
    h1 = jnp.dot(x, w1_ref[0], preferred_element_type=jnp.float32)
    h3 = jnp.dot(x, w3_ref[0], preferred_element_type=jnp.float32)
    h = (h1 * jax.nn.sigmoid(h1)) * h3
    contrib = jnp.dot(h, w2_ref[0], preferred_element_type=jnp.float32)
    scale = tkw_ref[b, k]

    @pl.when((k == 0) & (f == 0))
    def _init():
        out_ref[0] = scale * contrib

    @pl.when((k > 0) | (f > 0))
    def _acc():
        out_ref[0] += scale * contrib


def kernel(hidden_states, Wr, w1, w2, w3):
    x = hidden_states
    B, L, D = x.shape
    E = Wr.shape[1]
    FFN = w1.shape[2]
    K = 2
    FT = 512
    NF = FFN // FT

    logits, tkw, tki = pl.pallas_call(
        _router_body,
        out_shape=(
            jax.ShapeDtypeStruct((B, E), jnp.float32),
            jax.ShapeDtypeStruct((B, K), jnp.float32),
            jax.ShapeDtypeStruct((B, K), jnp.int32),
        ),
    )(x, Wr)

    grid_spec = pltpu.PrefetchScalarGridSpec(
        num_scalar_prefetch=2,
        grid=(B, K, NF),
        in_specs=[
            pl.BlockSpec((1, L, D), lambda b, k, f, ti, tw: (b, 0, 0)),
            pl.BlockSpec((1, D, FT), lambda b, k, f, ti, tw: (ti[b, k], 0, f)),
            pl.BlockSpec((1, D, FT), lambda b, k, f, ti, tw: (ti[b, k], 0, f)),
            pl.BlockSpec((1, FT, D), lambda b, k, f, ti, tw: (ti[b, k], f, 0)),
        ],
        out_specs=pl.BlockSpec((1, L, D), lambda b, k, f, ti, tw: (b, 0, 0)),
    )
    out = pl.pallas_call(
        _moe_body,
        grid_spec=grid_spec,
        out_shape=jax.ShapeDtypeStruct((B, L, D), jnp.float32),
    )(tki, tkw, x, w1, w3, w2)

    return (out, logits)
